# Initial kernel scaffold; baseline (speedup 1.0000x reference)
import jax
import jax.numpy as jnp
from jax import lax
from jax.experimental import pallas as pl
from jax.experimental.pallas import tpu as pltpu
from jax.experimental.pallas import tpu_sc as plsc

B = 16384
D = 128
NC = 2
NS = 16
L = 16
NW = NC * NS
BPW = B // NW
CH = 256


def _body(v_hbm, u_hbm, table_hbm, out_hbm,
          vidx, uidx, vrows, urows, res, tr, sem0, sem1):
    wid = lax.axis_index("s") * NC + lax.axis_index("c")
    base = wid * BPW
    pltpu.sync_copy(v_hbm.at[pl.ds(base, BPW)], vidx)
    pltpu.sync_copy(u_hbm.at[pl.ds(base, BPW)], uidx)

    @pl.loop(0, BPW, step=CH)
    def _chunk(c0):
        cpv = pltpu.async_copy(table_hbm.at[vidx.at[pl.ds(c0, CH)]], vrows, sem0)
        cpu_ = pltpu.async_copy(table_hbm.at[uidx.at[pl.ds(c0, CH)]], urows, sem1)
        cpv.wait()
        cpu_.wait()

        @pl.loop(0, CH, step=L)
        def _grp(p0):
            for i in range(L):
                acc = None
                for j in range(D // L):
                    vv = vrows[p0 + i, pl.ds(j * L, L)]
                    uu = urows[p0 + i, pl.ds(j * L, L)]
                    t = vv * uu
                    acc = t if acc is None else acc + t
                tr[i, :] = acc
            iota = lax.iota(jnp.int32, L)
            tot = None
            for j in range(L):
                col = plsc.load_gather(tr, [iota, jnp.full((L,), j, jnp.int32)])
                tot = col if tot is None else tot + col
            sig = 1.0 / (1.0 + jnp.exp(-tot))
            res[pl.ds(c0 + p0, L)] = sig

    pltpu.sync_copy(res, out_hbm.at[pl.ds(base, BPW)])


def kernel(pairs, kernel):
    table = kernel
    v = pairs[:, 0]
    u = pairs[:, 1]
    mesh = plsc.VectorSubcoreMesh(core_axis_name="c", subcore_axis_name="s")
    k = pl.kernel(
        _body,
        out_type=jax.ShapeDtypeStruct((B,), jnp.float32),
        mesh=mesh,
        scratch_types=[
            pltpu.VMEM((BPW,), jnp.int32),
            pltpu.VMEM((BPW,), jnp.int32),
            pltpu.VMEM((CH, D), jnp.float32),
            pltpu.VMEM((CH, D), jnp.float32),
            pltpu.VMEM((BPW,), jnp.float32),
            pltpu.VMEM((L, L), jnp.float32),
            pltpu.SemaphoreType.DMA,
            pltpu.SemaphoreType.DMA,
        ],
    )
    out = k(v, u, table)
    return out.reshape(B, 1)


# speedup vs baseline: 1.2176x; 1.2176x over previous
import dataclasses

import jax
import jax.numpy as jnp
from jax import lax
from jax.experimental import pallas as pl
from jax.experimental.pallas import tpu as pltpu
from jax.experimental.pallas import tpu_sc as plsc

B = 16384
D = 128
NC = 2
NS = 16
L = 16
NW = NC * NS
BPW = B // NW
CH = 256


def _body(v_hbm, u_hbm, table_hbm, out_hbm,
          vidx, uidx, vrows, urows, res, tr, sem0, sem1):
    wid = lax.axis_index("s") * NC + lax.axis_index("c")
    base = wid * BPW
    pltpu.sync_copy(v_hbm.at[pl.ds(base, BPW)], vidx)
    pltpu.sync_copy(u_hbm.at[pl.ds(base, BPW)], uidx)

    @pl.loop(0, BPW, step=CH)
    def _chunk(c0):
        cpv = pltpu.async_copy(table_hbm.at[vidx.at[pl.ds(c0, CH)]], vrows, sem0)
        cpu_ = pltpu.async_copy(table_hbm.at[uidx.at[pl.ds(c0, CH)]], urows, sem1)
        cpv.wait()
        cpu_.wait()

        @pl.loop(0, CH, step=L)
        def _grp(p0):
            for i in range(L):
                acc = None
                for j in range(D // L):
                    vv = vrows[p0 + i, pl.ds(j * L, L)]
                    uu = urows[p0 + i, pl.ds(j * L, L)]
                    t = vv * uu
                    acc = t if acc is None else acc + t
                tr[i, :] = acc
            iota = lax.iota(jnp.int32, L)
            tot = None
            for j in range(L):
                col = plsc.load_gather(tr, [iota, jnp.full((L,), j, jnp.int32)])
                tot = col if tot is None else tot + col
            sig = 1.0 / (1.0 + jnp.exp(-tot))
            res[pl.ds(c0 + p0, L)] = sig

    pltpu.sync_copy(res, out_hbm.at[pl.ds(base, BPW)])


def kernel(pairs, kernel):
    table = kernel
    v = pairs[:, 0]
    u = pairs[:, 1]
    mesh = plsc.VectorSubcoreMesh(core_axis_name="c", subcore_axis_name="s")
    cp = pltpu.CompilerParams()
    if "needs_layout_passes" in pltpu.CompilerParams.__dataclass_fields__:
        cp = dataclasses.replace(cp, needs_layout_passes=False)
    k = pl.kernel(
        _body,
        out_type=jax.ShapeDtypeStruct((B,), jnp.float32),
        mesh=mesh,
        scratch_types=[
            pltpu.VMEM((BPW,), jnp.int32),
            pltpu.VMEM((BPW,), jnp.int32),
            pltpu.VMEM((CH, D), jnp.float32),
            pltpu.VMEM((CH, D), jnp.float32),
            pltpu.VMEM((BPW,), jnp.float32),
            pltpu.VMEM((L, L), jnp.float32),
            pltpu.SemaphoreType.DMA,
            pltpu.SemaphoreType.DMA,
        ],
        compiler_params=cp,
    )
    out = k(v, u, table)
    return out.reshape(B, 1)


# speedup vs baseline: 1.2250x; 1.0061x over previous
import dataclasses

import jax
import jax.numpy as jnp
from jax import lax
from jax.experimental import pallas as pl
from jax.experimental.pallas import tpu as pltpu
from jax.experimental.pallas import tpu_sc as plsc

B = 16384
D = 128
NC = 2
NS = 16
L = 16
NW = NC * NS
BPW = B // NW
CH = 128
NCH = BPW // CH


def _body(v_hbm, u_hbm, table_hbm, out_hbm,
          vidx, uidx, vr0, ur0, vr1, ur1, res, tr, sem0, sem1):
    wid = lax.axis_index("s") * NC + lax.axis_index("c")
    base = wid * BPW
    pltpu.sync_copy(v_hbm.at[pl.ds(base, BPW)], vidx)
    pltpu.sync_copy(u_hbm.at[pl.ds(base, BPW)], uidx)

    slots = ((vr0, ur0, sem0), (vr1, ur1, sem1))

    def issue(c):
        vb, ub, sm = slots[c % 2]
        cv = pltpu.async_copy(table_hbm.at[vidx.at[pl.ds(c * CH, CH)]], vb, sm)
        cu = pltpu.async_copy(table_hbm.at[uidx.at[pl.ds(c * CH, CH)]], ub, sm)
        return (cv, cu)

    inflight = [issue(0), issue(1)]

    iota = lax.iota(jnp.int32, L)

    for c in range(NCH):
        vb, ub, _ = slots[c % 2]
        cv, cu = inflight[c]
        cv.wait()
        cu.wait()

        @pl.loop(0, CH, step=L)
        def _grp(p0, vb=vb, ub=ub, cbase=c * CH):
            for i in range(L):
                acc0 = None
                acc1 = None
                for j in range(0, D // L, 2):
                    t0 = vb[p0 + i, pl.ds(j * L, L)] * ub[p0 + i, pl.ds(j * L, L)]
                    t1 = vb[p0 + i, pl.ds((j + 1) * L, L)] * ub[p0 + i, pl.ds((j + 1) * L, L)]
                    acc0 = t0 if acc0 is None else acc0 + t0
                    acc1 = t1 if acc1 is None else acc1 + t1
                tr[i, :] = acc0 + acc1
            tot = None
            for j in range(L):
                col = plsc.load_gather(tr, [iota, jnp.full((L,), j, jnp.int32)])
                tot = col if tot is None else tot + col
            sig = 1.0 / (1.0 + jnp.exp(-tot))
            res[pl.ds(cbase + p0, L)] = sig

        if c + 2 < NCH:
            inflight.append(issue(c + 2))

    pltpu.sync_copy(res, out_hbm.at[pl.ds(base, BPW)])


def kernel(pairs, kernel):
    table = kernel
    v = pairs[:, 0]
    u = pairs[:, 1]
    mesh = plsc.VectorSubcoreMesh(core_axis_name="c", subcore_axis_name="s")
    cp = pltpu.CompilerParams()
    if "needs_layout_passes" in pltpu.CompilerParams.__dataclass_fields__:
        cp = dataclasses.replace(cp, needs_layout_passes=False)
    k = pl.kernel(
        _body,
        out_type=jax.ShapeDtypeStruct((B,), jnp.float32),
        mesh=mesh,
        scratch_types=[
            pltpu.VMEM((BPW,), jnp.int32),
            pltpu.VMEM((BPW,), jnp.int32),
            pltpu.VMEM((CH, D), jnp.float32),
            pltpu.VMEM((CH, D), jnp.float32),
            pltpu.VMEM((CH, D), jnp.float32),
            pltpu.VMEM((CH, D), jnp.float32),
            pltpu.VMEM((BPW,), jnp.float32),
            pltpu.VMEM((L, L), jnp.float32),
            pltpu.SemaphoreType.DMA,
            pltpu.SemaphoreType.DMA,
        ],
        compiler_params=cp,
    )
    out = k(v, u, table)
    return out.reshape(B, 1)


# speedup vs baseline: 1.3735x; 1.1212x over previous
import dataclasses

import jax
import jax.numpy as jnp
from jax import lax
from jax.experimental import pallas as pl
from jax.experimental.pallas import tpu as pltpu
from jax.experimental.pallas import tpu_sc as plsc

B = 16384
D = 128
NC = 2
NS = 16
L = 16
NW = NC * NS
BPW = B // NW
CH = 128
NCH = BPW // CH


def _body(v_hbm, u_hbm, table_hbm, out_hbm,
          vidx, uidx, vr0, ur0, vr1, ur1, res, tr, sem0, sem1):
    wid = lax.axis_index("s") * NC + lax.axis_index("c")
    base = wid * BPW
    pltpu.sync_copy(v_hbm.at[pl.ds(base, BPW)], vidx)
    pltpu.sync_copy(u_hbm.at[pl.ds(base, BPW)], uidx)

    slots = ((vr0, ur0, sem0), (vr1, ur1, sem1))

    def issue(c):
        vb, ub, sm = slots[c % 2]
        cv = pltpu.async_copy(table_hbm.at[vidx.at[pl.ds(c * CH, CH)]], vb, sm)
        cu = pltpu.async_copy(table_hbm.at[uidx.at[pl.ds(c * CH, CH)]], ub, sm)
        return (cv, cu)

    inflight = [issue(0), issue(1)]

    iota = lax.iota(jnp.int32, L)

    for c in range(NCH):
        vb, ub, _ = slots[c % 2]
        cv, cu = inflight[c]
        cv.wait()
        cu.wait()

        @pl.loop(0, CH, step=L)
        def _grp(p0, vb=vb, ub=ub, cbase=c * CH):
            DL = D // L

            def loads(i):
                vi = [vb[p0 + i, pl.ds(j * L, L)] for j in range(DL)]
                ui = [ub[p0 + i, pl.ds(j * L, L)] for j in range(DL)]
                return vi, ui

            def dot8(vs_us):
                vs, us = vs_us
                acc0 = vs[0] * us[0]
                acc1 = vs[1] * us[1]
                for j in range(2, DL, 2):
                    acc0 = acc0 + vs[j] * us[j]
                    acc1 = acc1 + vs[j + 1] * us[j + 1]
                return acc0 + acc1

            prev = loads(0)
            for i in range(1, L):
                cur = loads(i)
                tr[i - 1, :] = dot8(prev)
                prev = cur
            tr[L - 1, :] = dot8(prev)
            tot = None
            for j in range(L):
                col = plsc.load_gather(tr, [iota, jnp.full((L,), j, jnp.int32)])
                tot = col if tot is None else tot + col
            sig = 1.0 / (1.0 + jnp.exp(-tot))
            res[pl.ds(cbase + p0, L)] = sig

        if c + 2 < NCH:
            inflight.append(issue(c + 2))

    pltpu.sync_copy(res, out_hbm.at[pl.ds(base, BPW)])


def kernel(pairs, kernel):
    table = kernel
    v = pairs[:, 0]
    u = pairs[:, 1]
    mesh = plsc.VectorSubcoreMesh(core_axis_name="c", subcore_axis_name="s")
    cp = pltpu.CompilerParams()
    if "needs_layout_passes" in pltpu.CompilerParams.__dataclass_fields__:
        cp = dataclasses.replace(cp, needs_layout_passes=False)
    k = pl.kernel(
        _body,
        out_type=jax.ShapeDtypeStruct((B,), jnp.float32),
        mesh=mesh,
        scratch_types=[
            pltpu.VMEM((BPW,), jnp.int32),
            pltpu.VMEM((BPW,), jnp.int32),
            pltpu.VMEM((CH, D), jnp.float32),
            pltpu.VMEM((CH, D), jnp.float32),
            pltpu.VMEM((CH, D), jnp.float32),
            pltpu.VMEM((CH, D), jnp.float32),
            pltpu.VMEM((BPW,), jnp.float32),
            pltpu.VMEM((L, L), jnp.float32),
            pltpu.SemaphoreType.DMA,
            pltpu.SemaphoreType.DMA,
        ],
        compiler_params=cp,
    )
    out = k(v, u, table)
    return out.reshape(B, 1)
